# baseline (speedup 1.0000x reference)
import functools

import jax
import jax.numpy as jnp
from jax import lax
from jax.experimental import pallas as pl
from jax.experimental.pallas import tpu as pltpu
from jax.experimental.pallas import tpu_sc as plsc

B = 32
S = 8
MAX_K = 32
V = 100000
TEMPERATURE = 10.0

R = B * S
NC = 2
NS = 16
NW = NC * NS
RW = R // NW
L = 16
E = RW * MAX_K
CW = 4992
NB = 3
NCH = 20
TB = NCH * CW
TAIL = V - TB


def _body(vals_hbm, dist_hbm, out_hbm, vals_v, dist_v, wbuf, kbuf, bufs, tailb, sem):
    wid = lax.axis_index("s") * NC + lax.axis_index("c")
    base = wid * RW

    pltpu.sync_copy(vals_hbm.at[pl.ds(base * MAX_K, E)], vals_v)
    pltpu.sync_copy(dist_hbm.at[pl.ds(base * MAX_K, E)], dist_v)

    lane_iota = lax.iota(jnp.int32, L)
    zeros16 = jnp.zeros((L,), jnp.float32)
    zi = jnp.zeros((L,), jnp.int32)

    for bank in range(NB):
        for r in range(RW):
            def zstep(i, carry, bank=bank, r=r):
                for j in range(16):
                    bufs[bank, r, pl.ds(i * (16 * L) + j * L, L)] = zeros16
                return carry

            lax.fori_loop(0, CW // (16 * L), zstep, 0)
    for r in range(RW):
        for q in range(TAIL // L):
            tailb[r, pl.ds(q * L, L)] = zeros16

    for r in range(RW):
        v0 = vals_v[pl.ds(r * MAX_K, L)]
        v1 = vals_v[pl.ds(r * MAX_K + L, L)]
        d0 = dist_v[pl.ds(r * MAX_K, L)]
        d1 = dist_v[pl.ds(r * MAX_K + L, L)]

        e0 = jnp.exp(d0 * (-1.0 / TEMPERATURE))
        e1 = jnp.exp(d1 * (-1.0 / TEMPERATURE))
        t = e0 + e1
        for sh in (8, 4, 2, 1):
            t = t + t.at[lane_iota ^ sh].get(mode="promise_in_bounds")
        inv = 1.0 / t
        w0 = e0 * inv
        w1 = e1 * inv

        def mk_step(vsrc, wsrc, off):
            def step(j, carry):
                t0, t1, a0, a1 = carry
                jv = zi + j
                bv = vsrc.at[jv].get(mode="promise_in_bounds")
                bw = wsrc.at[jv].get(mode="promise_in_bounds")
                m0 = v0 == bv
                m1 = v1 == bv
                gj = j + off
                t0 = t0 + jnp.where(m0, bw, 0.0)
                t1 = t1 + jnp.where(m1, bw, 0.0)
                a0 = a0 + jnp.where(m0 & (gj < lane_iota), 1, 0)
                a1 = a1 + jnp.where(m1 & (gj < lane_iota + L), 1, 0)
                return t0, t1, a0, a1

            return step

        t0, t1, a0, a1 = lax.fori_loop(
            0, L, mk_step(v0, w0, 0), (zeros16, zeros16, zi, zi)
        )
        t0, t1, a0, a1 = lax.fori_loop(
            0, L, mk_step(v1, w1, L), (t0, t1, a0, a1)
        )
        wbuf[r, pl.ds(0, L)] = t0
        wbuf[r, pl.ds(L, L)] = t1
        kbuf[r, pl.ds(0, L)] = jnp.where(a0 == 0, 1, 0)
        kbuf[r, pl.ds(L, L)] = jnp.where(a1 == 0, 1, 0)

    def sweep(bank, c0, reset):
        for r in range(RW):
            v0 = vals_v[pl.ds(r * MAX_K, L)]
            v1 = vals_v[pl.ds(r * MAX_K + L, L)]
            t0 = wbuf[r, pl.ds(0, L)]
            t1 = wbuf[r, pl.ds(L, L)]
            k0 = kbuf[r, pl.ds(0, L)] > 0
            k1 = kbuf[r, pl.ds(L, L)] > 0
            rv = zi + r
            bv = zi + bank
            for vv, tt, kk in ((v0, t0, k0), (v1, t1, k1)):
                off = vv - c0
                inm = (off >= 0) & (off < CW) & kk
                idx = jnp.where(inm, off, 0)
                if reset:
                    plsc.store_scatter(bufs, [bv, rv, idx], zeros16, mask=inm)
                else:
                    plsc.addupdate_scatter(bufs, [bv, rv, idx], tt, mask=inm)

    copies = []
    for i in range(NCH):
        bank = i % NB
        c0 = i * CW
        if i >= NB:
            copies[i - NB].wait()
            sweep(bank, (i - NB) * CW, True)
        sweep(bank, c0, False)
        copies.append(
            pltpu.async_copy(
                bufs.at[bank],
                out_hbm.at[pl.ds(8 * wid, 8), pl.ds(c0, CW)],
                sem,
            )
        )

    for r in range(RW):
        v0 = vals_v[pl.ds(r * MAX_K, L)]
        v1 = vals_v[pl.ds(r * MAX_K + L, L)]
        t0 = wbuf[r, pl.ds(0, L)]
        t1 = wbuf[r, pl.ds(L, L)]
        k0 = kbuf[r, pl.ds(0, L)] > 0
        k1 = kbuf[r, pl.ds(L, L)] > 0
        for vv, tt, kk in ((v0, t0, k0), (v1, t1, k1)):
            off = vv - TB
            inm = (off >= 0) & (off < TAIL) & kk
            idx = jnp.where(inm, off, 0)
            plsc.addupdate_scatter(tailb, [zi + r, idx], tt, mask=inm)
    for r in range(RW):
        copies.append(
            pltpu.async_copy(
                tailb.at[r, pl.ds(0, 128)],
                out_hbm.at[base + r].at[pl.ds(TB, 128)],
                sem,
            )
        )
        copies.append(
            pltpu.async_copy(
                tailb.at[r, pl.ds(128, TAIL - 128)],
                out_hbm.at[base + r].at[pl.ds(TB + 128, TAIL - 128)],
                sem,
            )
        )
    for cp in copies[NCH - NB:]:
        cp.wait()


_combine = functools.partial(
    pl.kernel,
    mesh=plsc.VectorSubcoreMesh(core_axis_name="c", subcore_axis_name="s"),
    out_type=jax.ShapeDtypeStruct((R, V), jnp.float32),
    scratch_types=[
        pltpu.VMEM((E,), jnp.int32),
        pltpu.VMEM((E,), jnp.float32),
        pltpu.VMEM((RW, MAX_K), jnp.float32),
        pltpu.VMEM((RW, MAX_K), jnp.int32),
        pltpu.VMEM((NB, RW, CW), jnp.float32),
        pltpu.VMEM((RW, TAIL), jnp.float32),
        pltpu.SemaphoreType.DMA,
    ],
    compiler_params=pltpu.CompilerParams(needs_layout_passes=False),
)(_body)


def kernel(vals, distances):
    vals_flat = vals.reshape(R * MAX_K).astype(jnp.int32)
    dist_flat = distances.reshape(R * MAX_K).astype(jnp.float32)
    out = _combine(vals_flat, dist_flat)
    return out.reshape(B, S, V)


# speedup vs baseline: 1.1936x; 1.1936x over previous
import functools

import jax
import jax.numpy as jnp
from jax import lax
from jax.experimental import pallas as pl
from jax.experimental.pallas import tpu as pltpu
from jax.experimental.pallas import tpu_sc as plsc

B = 32
S = 8
MAX_K = 32
V = 100000
TEMPERATURE = 10.0

R = B * S
NC = 2
NS = 16
NW = NC * NS
ROWS_PER_W = R // NW
L = 16


def _body(vals_hbm, dist_hbm, out_hbm, vals_v, dist_v, row_buf, sem):
    wid = lax.axis_index("s") * NC + lax.axis_index("c")
    base = wid * ROWS_PER_W

    cp_v = pltpu.async_copy(
        vals_hbm.at[pl.ds(base * MAX_K, ROWS_PER_W * MAX_K)], vals_v, sem)
    cp_d = pltpu.async_copy(
        dist_hbm.at[pl.ds(base * MAX_K, ROWS_PER_W * MAX_K)], dist_v, sem)

    zeros16 = jnp.zeros((L,), jnp.float32)

    def zero_step(i, carry):
        for j in range(25):
            row_buf[pl.ds(i * (25 * L) + j * L, L)] = zeros16
        return carry

    lax.fori_loop(0, V // (25 * L), zero_step, 0)
    cp_v.wait()
    cp_d.wait()

    lane_iota = lax.iota(jnp.int32, L)

    for r in range(ROWS_PER_W):
        idx0 = vals_v[pl.ds(r * MAX_K, L)]
        idx1 = vals_v[pl.ds(r * MAX_K + L, L)]
        d0 = dist_v[pl.ds(r * MAX_K, L)]
        d1 = dist_v[pl.ds(r * MAX_K + L, L)]

        e0 = jnp.exp(d0 * (-1.0 / TEMPERATURE))
        e1 = jnp.exp(d1 * (-1.0 / TEMPERATURE))
        t = e0 + e1
        for sh in (8, 4, 2, 1):
            t = t + t.at[lane_iota ^ sh].get(mode="promise_in_bounds")
        inv = 1.0 / t
        w0 = e0 * inv
        w1 = e1 * inv

        for k in range(L):
            m = lane_iota == k
            plsc.addupdate_scatter(row_buf, [idx0], w0, mask=m)
        for k in range(L):
            m = lane_iota == k
            plsc.addupdate_scatter(row_buf, [idx1], w1, mask=m)

        pltpu.sync_copy(row_buf, out_hbm.at[base + r])
        plsc.store_scatter(row_buf, [idx0], zeros16)
        plsc.store_scatter(row_buf, [idx1], zeros16)


@functools.partial(
    pl.kernel,
    mesh=plsc.VectorSubcoreMesh(core_axis_name="c", subcore_axis_name="s"),
    out_type=jax.ShapeDtypeStruct((R, V), jnp.float32),
    scratch_types=[
        pltpu.VMEM((ROWS_PER_W * MAX_K,), jnp.int32),
        pltpu.VMEM((ROWS_PER_W * MAX_K,), jnp.float32),
        pltpu.VMEM((V,), jnp.float32),
        pltpu.SemaphoreType.DMA,
    ],
    compiler_params=pltpu.CompilerParams(needs_layout_passes=False),
)
def _combine(vals_hbm, dist_hbm, out_hbm, vals_v, dist_v, row_buf, sem):
    _body(vals_hbm, dist_hbm, out_hbm, vals_v, dist_v, row_buf, sem)


def kernel(vals, distances):
    vals_flat = vals.reshape(R * MAX_K).astype(jnp.int32)
    dist_flat = distances.reshape(R * MAX_K).astype(jnp.float32)
    out = _combine(vals_flat, dist_flat)
    return out.reshape(B, S, V)


# speedup vs baseline: 1.1939x; 1.0002x over previous
import functools

import jax
import jax.numpy as jnp
from jax import lax
from jax.experimental import pallas as pl
from jax.experimental.pallas import tpu as pltpu
from jax.experimental.pallas import tpu_sc as plsc

B = 32
S = 8
MAX_K = 32
V = 100000
TEMPERATURE = 10.0

R = B * S
NC = 2
NS = 16
NW = NC * NS
ROWS_PER_W = R // NW
L = 16


def _body(vals_hbm, dist_hbm, out_hbm, vals_v, dist_v, row_buf, sem):
    wid = lax.axis_index("s") * NC + lax.axis_index("c")
    base = wid * ROWS_PER_W

    cp_v = pltpu.async_copy(
        vals_hbm.at[pl.ds(base * MAX_K, ROWS_PER_W * MAX_K)], vals_v, sem)
    cp_d = pltpu.async_copy(
        dist_hbm.at[pl.ds(base * MAX_K, ROWS_PER_W * MAX_K)], dist_v, sem)

    zeros16 = jnp.zeros((L,), jnp.float32)

    def zero_step(i, carry):
        for j in range(25):
            row_buf[0, pl.ds(i * (25 * L) + j * L, L)] = zeros16
        return carry

    lax.fori_loop(0, V // (25 * L), zero_step, 0)
    cp_v.wait()
    cp_d.wait()

    lane_iota = lax.iota(jnp.int32, L)

    for r in range(ROWS_PER_W):
        idx0 = vals_v[pl.ds(r * MAX_K, L)]
        idx1 = vals_v[pl.ds(r * MAX_K + L, L)]
        d0 = dist_v[pl.ds(r * MAX_K, L)]
        d1 = dist_v[pl.ds(r * MAX_K + L, L)]

        e0 = jnp.exp(d0 * (-1.0 / TEMPERATURE))
        e1 = jnp.exp(d1 * (-1.0 / TEMPERATURE))
        t = e0 + e1
        for sh in (8, 4, 2, 1):
            t = t + t.at[lane_iota ^ sh].get(mode="promise_in_bounds")
        inv = 1.0 / t
        w0 = e0 * inv
        w1 = e1 * inv

        zidx = jnp.zeros((L,), jnp.int32)
        for k in range(L):
            m = lane_iota == k
            plsc.addupdate_scatter(row_buf, [zidx, idx0], w0, mask=m)
        for k in range(L):
            m = lane_iota == k
            plsc.addupdate_scatter(row_buf, [zidx, idx1], w1, mask=m)

        qcp = []
        for q0, qw in ((0, 24960), (24960, 24960), (49920, 24960),
                       (74880, V - 74880)):
            qcp.append(
                pltpu.async_copy(
                    row_buf.at[0, pl.ds(q0, qw)],
                    out_hbm.at[base + r].at[pl.ds(q0, qw)],
                    sem,
                )
            )
        for cp in qcp:
            cp.wait()
        plsc.store_scatter(row_buf, [zidx, idx0], zeros16)
        plsc.store_scatter(row_buf, [zidx, idx1], zeros16)


@functools.partial(
    pl.kernel,
    mesh=plsc.VectorSubcoreMesh(core_axis_name="c", subcore_axis_name="s"),
    out_type=jax.ShapeDtypeStruct((R, V), jnp.float32),
    scratch_types=[
        pltpu.VMEM((ROWS_PER_W * MAX_K,), jnp.int32),
        pltpu.VMEM((ROWS_PER_W * MAX_K,), jnp.float32),
        pltpu.VMEM((1, V), jnp.float32),
        pltpu.SemaphoreType.DMA,
    ],
    compiler_params=pltpu.CompilerParams(needs_layout_passes=False),
)
def _combine(vals_hbm, dist_hbm, out_hbm, vals_v, dist_v, row_buf, sem):
    _body(vals_hbm, dist_hbm, out_hbm, vals_v, dist_v, row_buf, sem)


def kernel(vals, distances):
    vals_flat = vals.reshape(R * MAX_K).astype(jnp.int32)
    dist_flat = distances.reshape(R * MAX_K).astype(jnp.float32)
    out = _combine(vals_flat, dist_flat)
    return out.reshape(B, S, V)


# speedup vs baseline: 1.1956x; 1.0014x over previous
import functools

import jax
import jax.numpy as jnp
from jax import lax
from jax.experimental import pallas as pl
from jax.experimental.pallas import tpu as pltpu
from jax.experimental.pallas import tpu_sc as plsc

B = 32
S = 8
MAX_K = 32
V = 100000
TEMPERATURE = 10.0

R = B * S
NC = 2
NS = 16
NW = NC * NS
ROWS_PER_W = R // NW
L = 16


def _body(vals_hbm, dist_hbm, out_hbm, vals_v, dist_v, row_buf, sem):
    wid = lax.axis_index("s") * NC + lax.axis_index("c")
    base = wid * ROWS_PER_W

    cp_v = pltpu.async_copy(
        vals_hbm.at[pl.ds(base * MAX_K, ROWS_PER_W * MAX_K)], vals_v, sem)
    cp_d = pltpu.async_copy(
        dist_hbm.at[pl.ds(base * MAX_K, ROWS_PER_W * MAX_K)], dist_v, sem)

    zeros16 = jnp.zeros((L,), jnp.float32)

    def zero_step(i, carry):
        for j in range(25):
            row_buf[pl.ds(i * (25 * L) + j * L, L)] = zeros16
        return carry

    lax.fori_loop(0, V // (25 * L), zero_step, 0)
    cp_v.wait()
    cp_d.wait()

    lane_iota = lax.iota(jnp.int32, L)

    for r in range(ROWS_PER_W):
        idx0 = vals_v[pl.ds(r * MAX_K, L)]
        idx1 = vals_v[pl.ds(r * MAX_K + L, L)]
        d0 = dist_v[pl.ds(r * MAX_K, L)]
        d1 = dist_v[pl.ds(r * MAX_K + L, L)]

        e0 = jnp.exp(d0 * (-1.0 / TEMPERATURE))
        e1 = jnp.exp(d1 * (-1.0 / TEMPERATURE))
        t = e0 + e1
        for sh in (8, 4, 2, 1):
            t = t + t.at[lane_iota ^ sh].get(mode="promise_in_bounds")
        inv = 1.0 / t
        w0 = e0 * inv
        w1 = e1 * inv

        for k in range(L):
            m = lane_iota == k
            plsc.addupdate_scatter(row_buf, [idx0], w0, mask=m)
        for k in range(L):
            m = lane_iota == k
            plsc.addupdate_scatter(row_buf, [idx1], w1, mask=m)

        pltpu.sync_copy(row_buf, out_hbm.at[base + r])
        plsc.store_scatter(row_buf, [idx0], zeros16)
        plsc.store_scatter(row_buf, [idx1], zeros16)


@functools.partial(
    pl.kernel,
    mesh=plsc.VectorSubcoreMesh(core_axis_name="c", subcore_axis_name="s"),
    out_type=jax.ShapeDtypeStruct((R, V), jnp.float32),
    scratch_types=[
        pltpu.VMEM((ROWS_PER_W * MAX_K,), jnp.int32),
        pltpu.VMEM((ROWS_PER_W * MAX_K,), jnp.float32),
        pltpu.VMEM((V,), jnp.float32),
        pltpu.SemaphoreType.DMA,
    ],
    compiler_params=pltpu.CompilerParams(needs_layout_passes=False),
)
def _combine(vals_hbm, dist_hbm, out_hbm, vals_v, dist_v, row_buf, sem):
    _body(vals_hbm, dist_hbm, out_hbm, vals_v, dist_v, row_buf, sem)


def kernel(vals, distances):
    vals_flat = vals.reshape(R * MAX_K).astype(jnp.int32)
    dist_flat = distances.reshape(R * MAX_K).astype(jnp.float32)
    out = _combine(vals_flat, dist_flat)
    return out.reshape(B, S, V)
